# Initial kernel scaffold; baseline (speedup 1.0000x reference)
import jax
import jax.numpy as jnp
from jax.experimental import pallas as pl


def kernel(x, edge_index, edge_attr, W_query, b_query, W_key, b_key, W_value, b_value, W_edge, W_msg_update, b_msg_update, W_msg, b_msg, ln_msg_g, ln_msg_b, ln_alpha_g, ln_alpha_b, W_concate, b_concate, bn_g, bn_b, W_skip, b_skip):
    raise NotImplementedError("write your pallas kernel here")


# speedup vs baseline: 1.7090x; 1.7090x over previous
import functools
import math

import jax
import jax.numpy as jnp
from jax.experimental import pallas as pl
from jax.experimental.pallas import tpu as pltpu

N = 10000
E = 160000
D = 128
C = 128

_NB = 10
_BN = N // _NB
_EB = 250
_BE = E // _EB


def _prologue_body(x_ref, wq_ref, bq_ref, wk_ref, bk_ref, wv_ref, bv_ref,
                   td_ref, ts_ref):
    x = x_ref[...]
    q = jnp.dot(x, wq_ref[...], preferred_element_type=jnp.float32) + bq_ref[...]
    k = jnp.dot(x, wk_ref[...], preferred_element_type=jnp.float32) + bk_ref[...]
    v = jnp.dot(x, wv_ref[...], preferred_element_type=jnp.float32) + bv_ref[...]
    td_ref[...] = jnp.concatenate([q, q * k, v], axis=1)
    ts_ref[...] = jnp.concatenate([k, v], axis=1)


def _prologue(x, W_query, b_query, W_key, b_key, W_value, b_value, interpret=False):
    full = lambda shape: pl.BlockSpec(shape, lambda i: (0, 0))
    return pl.pallas_call(
        _prologue_body,
        grid=(_NB,),
        in_specs=[
            pl.BlockSpec((_BN, D), lambda i: (i, 0)),
            full((D, C)), full((1, C)),
            full((D, C)), full((1, C)),
            full((D, C)), full((1, C)),
        ],
        out_specs=[
            pl.BlockSpec((_BN, 3 * C), lambda i: (i, 0)),
            pl.BlockSpec((_BN, 2 * C), lambda i: (i, 0)),
        ],
        out_shape=[
            jax.ShapeDtypeStruct((N, 3 * C), jnp.float32),
            jax.ShapeDtypeStruct((N, 2 * C), jnp.float32),
        ],
        interpret=interpret,
    )(x, W_query, b_query.reshape(1, C), W_key, b_key.reshape(1, C),
      W_value, b_value.reshape(1, C))


def _edge_body(ea_ref, gd_ref, gs_ref, wedge_ref, wmu_ref, bmu_ref,
               wmsg_ref, bmsg_ref, lag_ref, lab_ref, lmg_ref, lmb_ref,
               z_ref):
    ea = ea_ref[...]
    e = jnp.dot(ea, wedge_ref[...], preferred_element_type=jnp.float32)
    gd = gd_ref[...]
    gs = gs_ref[...]
    q_i = gd[:, 0:C]
    qk_i = gd[:, C:2 * C]
    v_i = gd[:, 2 * C:3 * C]
    k_j = gs[:, 0:C]
    v_j = gs[:, C:2 * C]
    scale = 1.0 / math.sqrt(3.0 * C)
    alpha = jnp.concatenate([qk_i, q_i * k_j, q_i * e], axis=1) * scale
    m = jnp.mean(alpha, axis=1, keepdims=True)
    var = jnp.mean(alpha * alpha, axis=1, keepdims=True) - m * m
    gate = jax.nn.sigmoid((alpha - m) * jax.lax.rsqrt(var + 1e-5) * lag_ref[...]
                          + lab_ref[...])
    msg = jnp.concatenate([v_i, v_j, e], axis=1)
    upd = jnp.dot(msg, wmu_ref[...], preferred_element_type=jnp.float32) + bmu_ref[...]
    t = upd * gate
    z = jnp.dot(t, wmsg_ref[...], preferred_element_type=jnp.float32) + bmsg_ref[...]
    zm = jnp.mean(z, axis=1, keepdims=True)
    zv = jnp.mean(z * z, axis=1, keepdims=True) - zm * zm
    z_ref[...] = (z - zm) * jax.lax.rsqrt(zv + 1e-5) * lmg_ref[...] + lmb_ref[...]


def _edge_stage(edge_attr, G_dst, G_src, W_edge, W_msg_update, b_msg_update,
                W_msg, b_msg, ln_alpha_g, ln_alpha_b, ln_msg_g, ln_msg_b,
                interpret=False):
    full = lambda shape: pl.BlockSpec(shape, lambda i: (0, 0))
    return pl.pallas_call(
        _edge_body,
        grid=(_EB,),
        in_specs=[
            pl.BlockSpec((_BE, 16), lambda i: (i, 0)),
            pl.BlockSpec((_BE, 3 * C), lambda i: (i, 0)),
            pl.BlockSpec((_BE, 2 * C), lambda i: (i, 0)),
            full((16, C)),
            full((3 * C, 3 * C)), full((1, 3 * C)),
            full((3 * C, C)), full((1, C)),
            full((1, 3 * C)), full((1, 3 * C)),
            full((1, C)), full((1, C)),
        ],
        out_specs=pl.BlockSpec((_BE, C), lambda i: (i, 0)),
        out_shape=jax.ShapeDtypeStruct((E, C), jnp.float32),
        interpret=interpret,
    )(edge_attr, G_dst, G_src, W_edge, W_msg_update,
      b_msg_update.reshape(1, 3 * C), W_msg, b_msg.reshape(1, C),
      ln_alpha_g.reshape(1, 3 * C), ln_alpha_b.reshape(1, 3 * C),
      ln_msg_g.reshape(1, C), ln_msg_b.reshape(1, C))


def _epi1_body(agg_ref, wc_ref, bc_ref, out1_ref, ssum_ref, ssq_ref):
    i = pl.program_id(0)
    o = jnp.dot(agg_ref[...], wc_ref[...], preferred_element_type=jnp.float32) + bc_ref[...]
    out1_ref[...] = o
    s = jnp.sum(o, axis=0, keepdims=True)
    sq = jnp.sum(o * o, axis=0, keepdims=True)

    @pl.when(i == 0)
    def _():
        ssum_ref[...] = jnp.zeros_like(ssum_ref)
        ssq_ref[...] = jnp.zeros_like(ssq_ref)

    ssum_ref[...] += s
    ssq_ref[...] += sq


def _epi1(agg, W_concate, b_concate, interpret=False):
    full = lambda shape: pl.BlockSpec(shape, lambda i: (0, 0))
    return pl.pallas_call(
        _epi1_body,
        grid=(_NB,),
        in_specs=[
            pl.BlockSpec((_BN, C), lambda i: (i, 0)),
            full((C, C)), full((1, C)),
        ],
        out_specs=[
            pl.BlockSpec((_BN, C), lambda i: (i, 0)),
            full((1, C)), full((1, C)),
        ],
        out_shape=[
            jax.ShapeDtypeStruct((N, C), jnp.float32),
            jax.ShapeDtypeStruct((1, C), jnp.float32),
            jax.ShapeDtypeStruct((1, C), jnp.float32),
        ],
        interpret=interpret,
    )(agg, W_concate, b_concate.reshape(1, C))


def _epi2_body(out1_ref, ssum_ref, ssq_ref, x_ref, wskip_ref, bskip_ref,
               bng_ref, bnb_ref, out_ref):
    o = out1_ref[...]
    mean = ssum_ref[...] * (1.0 / N)
    var = ssq_ref[...] * (1.0 / N) - mean * mean
    o = bng_ref[...] * (o - mean) * jax.lax.rsqrt(var + 1e-5) + bnb_ref[...]
    o = o * jax.nn.sigmoid(o)
    skip = jnp.dot(x_ref[...], wskip_ref[...], preferred_element_type=jnp.float32) + bskip_ref[...]
    out_ref[...] = o + skip


def _epi2(out1, ssum, ssq, x, W_skip, b_skip, bn_g, bn_b, interpret=False):
    full = lambda shape: pl.BlockSpec(shape, lambda i: (0, 0))
    return pl.pallas_call(
        _epi2_body,
        grid=(_NB,),
        in_specs=[
            pl.BlockSpec((_BN, C), lambda i: (i, 0)),
            full((1, C)), full((1, C)),
            pl.BlockSpec((_BN, D), lambda i: (i, 0)),
            full((D, C)), full((1, C)),
            full((1, C)), full((1, C)),
        ],
        out_specs=pl.BlockSpec((_BN, C), lambda i: (i, 0)),
        out_shape=jax.ShapeDtypeStruct((N, C), jnp.float32),
        interpret=interpret,
    )(out1, ssum, ssq, x, W_skip, b_skip.reshape(1, C),
      bn_g.reshape(1, C), bn_b.reshape(1, C))


def kernel(x, edge_index, edge_attr, W_query, b_query, W_key, b_key,
           W_value, b_value, W_edge, W_msg_update, b_msg_update, W_msg,
           b_msg, ln_msg_g, ln_msg_b, ln_alpha_g, ln_alpha_b, W_concate,
           b_concate, bn_g, bn_b, W_skip, b_skip, interpret=False):
    src = edge_index[0]
    dst = edge_index[1]
    TD, TS = _prologue(x, W_query, b_query, W_key, b_key, W_value, b_value,
                       interpret=interpret)
    G_dst = TD[dst]
    G_src = TS[src]
    z = _edge_stage(edge_attr, G_dst, G_src, W_edge, W_msg_update,
                    b_msg_update, W_msg, b_msg, ln_alpha_g, ln_alpha_b,
                    ln_msg_g, ln_msg_b, interpret=interpret)
    agg = jax.ops.segment_sum(z, dst, num_segments=N)
    out1, ssum, ssq = _epi1(agg, W_concate, b_concate, interpret=interpret)
    return _epi2(out1, ssum, ssq, x, W_skip, b_skip, bn_g, bn_b,
                 interpret=interpret)


# speedup vs baseline: 3.3591x; 1.9656x over previous
import functools
import math

import jax
import jax.numpy as jnp
from jax import lax
from jax.experimental import pallas as pl
from jax.experimental.pallas import tpu as pltpu
from jax.experimental.pallas import tpu_sc as plsc

N = 10000
E = 160000
D = 128
C = 128

_NB = 10
_BN = N // _NB
_EB = 250
_BE = E // _EB

_NC = 2
_NS = 16
_NW = _NC * _NS
_CHUNK = 128
_NCHUNK = E // _CHUNK
_CPW = -(-_NCHUNK // _NW)
_U = 80
_NU = N // _U
_UPW = -(-_NU // _NS)


def _sc_gather_body(src_ref, dst_ref, td_ref, ts_ref, gd_ref, gs_ref,
                    idx_v, rows_d, rows_s, sem):
    c = lax.axis_index("c")
    s = lax.axis_index("s")
    wid = s * _NC + c

    def body(j, carry):
        ci = wid + j * _NW

        @pl.when(ci < _NCHUNK)
        def _():
            off = ci * _CHUNK
            pltpu.sync_copy(dst_ref.at[pl.ds(off, _CHUNK)], idx_v)
            pltpu.async_copy(td_ref.at[idx_v], rows_d, sem).wait()
            pltpu.sync_copy(rows_d, gd_ref.at[pl.ds(off, _CHUNK)])
            pltpu.sync_copy(src_ref.at[pl.ds(off, _CHUNK)], idx_v)
            pltpu.async_copy(ts_ref.at[idx_v], rows_s, sem).wait()
            pltpu.sync_copy(rows_s, gs_ref.at[pl.ds(off, _CHUNK)])

        return carry

    lax.fori_loop(0, _CPW, body, 0)


def _sc_gather(src, dst, TD, TS):
    mesh = plsc.VectorSubcoreMesh(core_axis_name="c", subcore_axis_name="s")
    return pl.kernel(
        _sc_gather_body,
        out_type=[
            jax.ShapeDtypeStruct((E, 3 * C), jnp.float32),
            jax.ShapeDtypeStruct((E, 2 * C), jnp.float32),
        ],
        mesh=mesh,
        scratch_types=[
            pltpu.VMEM((_CHUNK,), jnp.int32),
            pltpu.VMEM((_CHUNK, 3 * C), jnp.float32),
            pltpu.VMEM((_CHUNK, 2 * C), jnp.float32),
            pltpu.SemaphoreType.DMA,
        ],
    )(src, dst, TD, TS)


def _sc_scatter_body(z_ref, dst_ref, out_ref,
                     idx_v, z_v, stage_v, acc_shared, sem):
    c = lax.axis_index("c")
    s = lax.axis_index("s")
    wid = s * _NC + c

    def zbody(i, carry):
        r = i // 8
        l = i % 8
        stage_v[r, pl.ds(l * 16, 16)] = jnp.zeros((16,), jnp.float32)
        return carry

    lax.fori_loop(0, _U * 8, zbody, 0)

    def ubody(j, carry):
        u = s + j * _NS

        @pl.when(u < _NU)
        def _():
            pltpu.sync_copy(stage_v, acc_shared.at[pl.ds(u * _U, _U)])

        return carry

    lax.fori_loop(0, _UPW, ubody, 0)
    plsc.subcore_barrier()

    def body(j, carry):
        ci = wid + j * _NW

        @pl.when(ci < _NCHUNK)
        def _():
            off = ci * _CHUNK
            pltpu.sync_copy(dst_ref.at[pl.ds(off, _CHUNK)], idx_v)
            pltpu.sync_copy(z_ref.at[pl.ds(off, _CHUNK)], z_v)
            pltpu.sync_copy(z_v, acc_shared.at[idx_v], add=True)

        return carry

    lax.fori_loop(0, _CPW, body, 0)
    plsc.subcore_barrier()

    def dbody(j, carry):
        u = s + j * _NS

        @pl.when(u < _NU)
        def _():
            pltpu.sync_copy(acc_shared.at[pl.ds(u * _U, _U)], stage_v)
            pltpu.sync_copy(stage_v, out_ref.at[c, pl.ds(u * _U, _U)])

        return carry

    lax.fori_loop(0, _UPW, dbody, 0)


def _sc_scatter(z, dst):
    mesh = plsc.VectorSubcoreMesh(core_axis_name="c", subcore_axis_name="s")
    return pl.kernel(
        _sc_scatter_body,
        out_type=jax.ShapeDtypeStruct((_NC, N, C), jnp.float32),
        mesh=mesh,
        scratch_types=[
            pltpu.VMEM((_CHUNK,), jnp.int32),
            pltpu.VMEM((_CHUNK, C), jnp.float32),
            pltpu.VMEM((_U, C), jnp.float32),
            pltpu.VMEM_SHARED((N, C), jnp.float32),
            pltpu.SemaphoreType.DMA,
        ],
    )(z, dst)


def _prologue_body(x_ref, wq_ref, bq_ref, wk_ref, bk_ref, wv_ref, bv_ref,
                   td_ref, ts_ref):
    x = x_ref[...]
    q = jnp.dot(x, wq_ref[...], preferred_element_type=jnp.float32) + bq_ref[...]
    k = jnp.dot(x, wk_ref[...], preferred_element_type=jnp.float32) + bk_ref[...]
    v = jnp.dot(x, wv_ref[...], preferred_element_type=jnp.float32) + bv_ref[...]
    td_ref[...] = jnp.concatenate([q, q * k, v], axis=1)
    ts_ref[...] = jnp.concatenate([k, v], axis=1)


def _prologue(x, W_query, b_query, W_key, b_key, W_value, b_value, interpret=False):
    full = lambda shape: pl.BlockSpec(shape, lambda i: (0, 0))
    return pl.pallas_call(
        _prologue_body,
        grid=(_NB,),
        in_specs=[
            pl.BlockSpec((_BN, D), lambda i: (i, 0)),
            full((D, C)), full((1, C)),
            full((D, C)), full((1, C)),
            full((D, C)), full((1, C)),
        ],
        out_specs=[
            pl.BlockSpec((_BN, 3 * C), lambda i: (i, 0)),
            pl.BlockSpec((_BN, 2 * C), lambda i: (i, 0)),
        ],
        out_shape=[
            jax.ShapeDtypeStruct((N, 3 * C), jnp.float32),
            jax.ShapeDtypeStruct((N, 2 * C), jnp.float32),
        ],
        interpret=interpret,
    )(x, W_query, b_query.reshape(1, C), W_key, b_key.reshape(1, C),
      W_value, b_value.reshape(1, C))


def _edge_body(ea_ref, gd_ref, gs_ref, wedge_ref, wmu_ref, bmu_ref,
               wmsg_ref, bmsg_ref, lag_ref, lab_ref, lmg_ref, lmb_ref,
               z_ref):
    ea = ea_ref[...]
    e = jnp.dot(ea, wedge_ref[...], preferred_element_type=jnp.float32)
    gd = gd_ref[...]
    gs = gs_ref[...]
    q_i = gd[:, 0:C]
    qk_i = gd[:, C:2 * C]
    v_i = gd[:, 2 * C:3 * C]
    k_j = gs[:, 0:C]
    v_j = gs[:, C:2 * C]
    scale = 1.0 / math.sqrt(3.0 * C)
    alpha = jnp.concatenate([qk_i, q_i * k_j, q_i * e], axis=1) * scale
    m = jnp.mean(alpha, axis=1, keepdims=True)
    var = jnp.mean(alpha * alpha, axis=1, keepdims=True) - m * m
    gate = jax.nn.sigmoid((alpha - m) * jax.lax.rsqrt(var + 1e-5) * lag_ref[...]
                          + lab_ref[...])
    msg = jnp.concatenate([v_i, v_j, e], axis=1)
    upd = jnp.dot(msg, wmu_ref[...], preferred_element_type=jnp.float32) + bmu_ref[...]
    t = upd * gate
    z = jnp.dot(t, wmsg_ref[...], preferred_element_type=jnp.float32) + bmsg_ref[...]
    zm = jnp.mean(z, axis=1, keepdims=True)
    zv = jnp.mean(z * z, axis=1, keepdims=True) - zm * zm
    z_ref[...] = (z - zm) * jax.lax.rsqrt(zv + 1e-5) * lmg_ref[...] + lmb_ref[...]


def _edge_stage(edge_attr, G_dst, G_src, W_edge, W_msg_update, b_msg_update,
                W_msg, b_msg, ln_alpha_g, ln_alpha_b, ln_msg_g, ln_msg_b,
                interpret=False):
    full = lambda shape: pl.BlockSpec(shape, lambda i: (0, 0))
    return pl.pallas_call(
        _edge_body,
        grid=(_EB,),
        in_specs=[
            pl.BlockSpec((_BE, 16), lambda i: (i, 0)),
            pl.BlockSpec((_BE, 3 * C), lambda i: (i, 0)),
            pl.BlockSpec((_BE, 2 * C), lambda i: (i, 0)),
            full((16, C)),
            full((3 * C, 3 * C)), full((1, 3 * C)),
            full((3 * C, C)), full((1, C)),
            full((1, 3 * C)), full((1, 3 * C)),
            full((1, C)), full((1, C)),
        ],
        out_specs=pl.BlockSpec((_BE, C), lambda i: (i, 0)),
        out_shape=jax.ShapeDtypeStruct((E, C), jnp.float32),
        interpret=interpret,
    )(edge_attr, G_dst, G_src, W_edge, W_msg_update,
      b_msg_update.reshape(1, 3 * C), W_msg, b_msg.reshape(1, C),
      ln_alpha_g.reshape(1, 3 * C), ln_alpha_b.reshape(1, 3 * C),
      ln_msg_g.reshape(1, C), ln_msg_b.reshape(1, C))


def _epi1_body(agg0_ref, agg1_ref, wc_ref, bc_ref, out1_ref, ssum_ref, ssq_ref):
    i = pl.program_id(0)
    agg = agg0_ref[...] + agg1_ref[...]
    o = jnp.dot(agg, wc_ref[...], preferred_element_type=jnp.float32) + bc_ref[...]
    out1_ref[...] = o
    s = jnp.sum(o, axis=0, keepdims=True)
    sq = jnp.sum(o * o, axis=0, keepdims=True)

    @pl.when(i == 0)
    def _():
        ssum_ref[...] = jnp.zeros_like(ssum_ref)
        ssq_ref[...] = jnp.zeros_like(ssq_ref)

    ssum_ref[...] += s
    ssq_ref[...] += sq


def _epi1(agg0, agg1, W_concate, b_concate, interpret=False):
    full = lambda shape: pl.BlockSpec(shape, lambda i: (0, 0))
    return pl.pallas_call(
        _epi1_body,
        grid=(_NB,),
        in_specs=[
            pl.BlockSpec((_BN, C), lambda i: (i, 0)),
            pl.BlockSpec((_BN, C), lambda i: (i, 0)),
            full((C, C)), full((1, C)),
        ],
        out_specs=[
            pl.BlockSpec((_BN, C), lambda i: (i, 0)),
            full((1, C)), full((1, C)),
        ],
        out_shape=[
            jax.ShapeDtypeStruct((N, C), jnp.float32),
            jax.ShapeDtypeStruct((1, C), jnp.float32),
            jax.ShapeDtypeStruct((1, C), jnp.float32),
        ],
        interpret=interpret,
    )(agg0, agg1, W_concate, b_concate.reshape(1, C))


def _epi2_body(out1_ref, ssum_ref, ssq_ref, x_ref, wskip_ref, bskip_ref,
               bng_ref, bnb_ref, out_ref):
    o = out1_ref[...]
    mean = ssum_ref[...] * (1.0 / N)
    var = ssq_ref[...] * (1.0 / N) - mean * mean
    o = bng_ref[...] * (o - mean) * jax.lax.rsqrt(var + 1e-5) + bnb_ref[...]
    o = o * jax.nn.sigmoid(o)
    skip = jnp.dot(x_ref[...], wskip_ref[...], preferred_element_type=jnp.float32) + bskip_ref[...]
    out_ref[...] = o + skip


def _epi2(out1, ssum, ssq, x, W_skip, b_skip, bn_g, bn_b, interpret=False):
    full = lambda shape: pl.BlockSpec(shape, lambda i: (0, 0))
    return pl.pallas_call(
        _epi2_body,
        grid=(_NB,),
        in_specs=[
            pl.BlockSpec((_BN, C), lambda i: (i, 0)),
            full((1, C)), full((1, C)),
            pl.BlockSpec((_BN, D), lambda i: (i, 0)),
            full((D, C)), full((1, C)),
            full((1, C)), full((1, C)),
        ],
        out_specs=pl.BlockSpec((_BN, C), lambda i: (i, 0)),
        out_shape=jax.ShapeDtypeStruct((N, C), jnp.float32),
        interpret=interpret,
    )(out1, ssum, ssq, x, W_skip, b_skip.reshape(1, C),
      bn_g.reshape(1, C), bn_b.reshape(1, C))


def kernel(x, edge_index, edge_attr, W_query, b_query, W_key, b_key,
           W_value, b_value, W_edge, W_msg_update, b_msg_update, W_msg,
           b_msg, ln_msg_g, ln_msg_b, ln_alpha_g, ln_alpha_b, W_concate,
           b_concate, bn_g, bn_b, W_skip, b_skip):
    src = edge_index[0]
    dst = edge_index[1]
    TD, TS = _prologue(x, W_query, b_query, W_key, b_key, W_value, b_value)
    G_dst, G_src = _sc_gather(src, dst, TD, TS)
    z = _edge_stage(edge_attr, G_dst, G_src, W_edge, W_msg_update,
                    b_msg_update, W_msg, b_msg, ln_alpha_g, ln_alpha_b,
                    ln_msg_g, ln_msg_b)
    parts = _sc_scatter(z, dst)
    out1, ssum, ssq = _epi1(parts[0], parts[1], W_concate, b_concate)
    return _epi2(out1, ssum, ssq, x, W_skip, b_skip, bn_g, bn_b)
